# baseline (speedup 1.0000x reference)
import functools

import jax
import jax.numpy as jnp
import numpy as np
from jax import lax
from jax.experimental import pallas as pl
from jax.experimental.pallas import tpu as pltpu
from jax.experimental.pallas import tpu_sc as plsc

C = 1.0
MIN = 1e-15
MAXEPS = 4e-3

NP = 10240
BM = 512
BN = 2048


def _artanh(x):
    return jnp.arctanh(jnp.clip(x, -1 + 1e-7, 1 - 1e-7))


def _proj(x):
    n = jnp.maximum(jnp.linalg.norm(x, axis=-1, keepdims=True), MIN)
    maxnorm = (1.0 - MAXEPS) / jnp.sqrt(C)
    return jnp.where(n > maxnorm, x / n * maxnorm, x)


def _expmap0(u):
    sc = jnp.sqrt(C)
    n = jnp.maximum(jnp.linalg.norm(u, axis=-1, keepdims=True), MIN)
    return jnp.tanh(sc * n) * u / (sc * n)


def _logmap0(p):
    sc = jnp.sqrt(C)
    n = jnp.maximum(jnp.linalg.norm(p, axis=-1, keepdims=True), MIN)
    return _artanh(sc * n) / (sc * n) * p


def _mobius_add(x, y):
    x2 = jnp.sum(x * x, -1, keepdims=True)
    y2 = jnp.sum(y * y, -1, keepdims=True)
    xy = jnp.sum(x * y, -1, keepdims=True)
    num = (1 + 2 * C * xy + C * y2) * x + (1 - C * x2) * y
    den = 1 + 2 * C * xy + C * C * x2 * y2
    return num / jnp.maximum(den, MIN)


def _mobius_matvec(m, x):
    sc = jnp.sqrt(C)
    xn = jnp.maximum(jnp.linalg.norm(x, axis=-1, keepdims=True), MIN)
    mx = x @ m.T
    mxn = jnp.maximum(jnp.linalg.norm(mx, axis=-1, keepdims=True), MIN)
    res = jnp.tanh(mxn / xn * _artanh(sc * xn)) * mx / (mxn * sc)
    zero = jnp.all(mx == 0, axis=-1, keepdims=True)
    return jnp.where(zero, jnp.zeros_like(res), res)


def _hgc_pre(x, W, b):
    mv = _proj(_mobius_matvec(W, x))
    hb = _proj(_expmap0(b))
    h = _proj(_mobius_add(mv, hb))
    return _logmap0(h)


def _hgc_post(sup):
    out = _proj(_expmap0(sup))
    return _logmap0(_proj(_expmap0(jax.nn.relu(_logmap0(out)))))


def _readout(x):
    return jnp.concatenate([jnp.max(x, axis=0, keepdims=True),
                            jnp.mean(x, axis=0, keepdims=True)], axis=1)


NPNP = NP * NP
HALF = NPNP // 2
ZCH = 32768
NZC = HALF // 16 // ZCH
ZBATCH = 10
NCH = 168
SBATCH = 24
EPW = NCH * 128


def _sc_scatter_body(idx_hbm, zsrc_hbm, tbl_hbm, idxv, idx2v, valv, zbuf,
                     sem):
    cid = lax.axis_index("c")
    sid = lax.axis_index("s")
    pltpu.sync_copy(zsrc_hbm, zbuf)
    pltpu.sync_copy(idx_hbm.at[sid], idxv)
    lo = cid * HALF
    lane = lax.iota(jnp.int32, 16)

    def mask_step(j, _):
        safe = lane * NP + (lo + 10000 + lax.rem(j, 200))
        for v in range(8):
            iv = idxv[pl.ds(j * 128 + v * 16, 16)]
            owned = (iv >= lo) & (iv < lo + HALF)
            idx2v[j, pl.ds(v * 16, 16)] = jnp.where(owned, iv, safe)
            valv[j, pl.ds(v * 16, 16)] = jnp.where(owned, 1.0, 0.0)
        return _

    lax.fori_loop(0, NCH, mask_step, 0)

    base = lo + sid * (HALF // 16)

    def zero_step(z, _):
        hs = []
        for b in range(ZBATCH):
            hs.append(pltpu.async_copy(
                zbuf, tbl_hbm.at[pl.ds(base + (z * ZBATCH + b) * ZCH, ZCH)],
                sem))
        for h in hs:
            h.wait()
        return _

    lax.fori_loop(0, NZC // ZBATCH, zero_step, 0)
    plsc.subcore_barrier()

    def scat_step(i, _):
        hs = []
        for b in range(SBATCH):
            j = i * SBATCH + b
            hs.append(pltpu.async_copy(
                valv.at[j], tbl_hbm.at[idx2v.at[j]], sem))
        for h in hs:
            h.wait()
        return _

    lax.fori_loop(0, NCH // SBATCH, scat_step, 0)


def _build_table(row, col):
    E = row.shape[0]
    idx = row * NP + col
    big = jnp.pad(idx, (0, 16 * EPW - E), constant_values=-1)
    idx2 = big.reshape(16, EPW)

    mesh = plsc.VectorSubcoreMesh(core_axis_name="c", subcore_axis_name="s")
    tbl = pl.kernel(
        _sc_scatter_body,
        out_type=jax.ShapeDtypeStruct((NPNP,), jnp.float32),
        mesh=mesh,
        scratch_types=[
            pltpu.VMEM((EPW,), jnp.int32),
            pltpu.VMEM((NCH, 128), jnp.int32),
            pltpu.VMEM((NCH, 128), jnp.float32),
            pltpu.VMEM((ZCH,), jnp.float32),
            pltpu.SemaphoreType.DMA,
        ],
    )(idx2, jnp.zeros((ZCH,), jnp.float32))
    return tbl.reshape(NP, NP)


def _mm_kernel(a_ref, b_ref, o_ref):
    o_ref[...] = jnp.dot(a_ref[...], b_ref[...],
                         preferred_element_type=jnp.float32)


def _pallas_mm(a, b, bm=256):
    n, k = a.shape
    d = b.shape[1]
    out = pl.pallas_call(
        _mm_kernel,
        grid=(n // bm,),
        in_specs=[
            pl.BlockSpec((bm, k), lambda i: (i, 0)),
            pl.BlockSpec((k, d), lambda i: (0, 0)),
        ],
        out_specs=pl.BlockSpec((bm, d), lambda i: (i, 0)),
        out_shape=jax.ShapeDtypeStruct((n, d), jnp.float32),
    )(a, b)
    return out


def _p1_kernel(t_ref, r_ref, k_ref, o_ref, d_ref, b_ref):
    j = pl.program_id(1)
    blk = t_ref[...]
    o = jnp.dot(blk, r_ref[...], preferred_element_type=jnp.float32)
    acc_d = jnp.zeros((BM, 128), jnp.float32)
    acc_b = jnp.zeros((BM, 128), jnp.float32)
    for t in range(16):
        sl = blk[:, t * 128:(t + 1) * 128]
        acc_d += sl * k_ref[:, t * 128:(t + 1) * 128]
        acc_b += sl * (2.0 ** t)
    b_ref[...] = acc_b

    @pl.when(j == 0)
    def _():
        o_ref[...] = o
        d_ref[...] = acc_d

    @pl.when(j > 0)
    def _():
        o_ref[...] += o
        d_ref[...] += acc_d


def _pass1(T, rhs, kcol):
    return pl.pallas_call(
        _p1_kernel,
        grid=(NP // BM, NP // BN),
        in_specs=[
            pl.BlockSpec((BM, BN), lambda i, j: (i, j)),
            pl.BlockSpec((BN, 128), lambda i, j: (j, 0)),
            pl.BlockSpec((1, BN), lambda i, j: (0, j)),
        ],
        out_specs=[
            pl.BlockSpec((BM, 128), lambda i, j: (i, 0)),
            pl.BlockSpec((BM, 128), lambda i, j: (i, 0)),
            pl.BlockSpec((BM, 128), lambda i, j: (i, j)),
        ],
        out_shape=[
            jax.ShapeDtypeStruct((NP, 128), jnp.float32),
            jax.ShapeDtypeStruct((NP, 128), jnp.float32),
            jax.ShapeDtypeStruct((NP, NP // 16), jnp.float32),
        ],
        compiler_params=pltpu.CompilerParams(
            dimension_semantics=("parallel", "arbitrary")),
    )(T, rhs, kcol)


def _p2_kernel(b_ref, z_ref, o_ref):
    j = pl.program_id(1)
    m = b_ref[...]
    acc = jnp.zeros((BM, 64), jnp.float32)
    for t in range(16):
        h = jnp.floor(m * 0.5)
        bit = m - 2.0 * h
        m = h
        acc += jnp.dot(bit, z_ref[pl.ds(t * 128, 128), :],
                       preferred_element_type=jnp.float32)

    @pl.when(j == 0)
    def _():
        o_ref[...] = acc

    @pl.when(j > 0)
    def _():
        o_ref[...] += acc


def _pass2(bmp, z):
    return pl.pallas_call(
        _p2_kernel,
        grid=(NP // BM, NP // BN),
        in_specs=[
            pl.BlockSpec((BM, 128), lambda i, j: (i, j)),
            pl.BlockSpec((BN, 64), lambda i, j: (j, 0)),
        ],
        out_specs=pl.BlockSpec((BM, 64), lambda i, j: (i, 0)),
        out_shape=jax.ShapeDtypeStruct((NP, 64), jnp.float32),
        compiler_params=pltpu.CompilerParams(
            dimension_semantics=("parallel", "arbitrary")),
    )(bmp, z)


def _f2_kernel(a_ref, s_ref, bm_ref, b_ref, sf_ref, k_ref, y_ref, o_ref):
    j = pl.program_id(1)
    a = a_ref[...]
    sr = s_ref[...]
    m = bm_ref[...]
    q = jnp.zeros((BM, 64), jnp.float32)
    p = jnp.zeros((BM, 1), jnp.float32)
    for t in range(16):
        h = jnp.floor(m * 0.5)
        bit = m - 2.0 * h
        m = h
        bt = b_ref[:, t * 128:(t + 1) * 128]
        st = sf_ref[:, t * 128:(t + 1) * 128]
        kt = k_ref[:, t * 128:(t + 1) * 128]
        w = bit * (sr + st) * 0.5
        E = kt * jnp.exp(jnp.maximum(a + bt, 0.0) + w)
        q += jnp.dot(E, y_ref[pl.ds(t * 128, 128), :],
                     preferred_element_type=jnp.float32)
        p += jnp.sum(E, axis=1, keepdims=True)
    blk = jnp.concatenate([q, jnp.broadcast_to(p, (BM, 64))], axis=1)

    @pl.when(j == 0)
    def _():
        o_ref[...] = blk

    @pl.when(j > 0)
    def _():
        o_ref[...] += blk


def _flash2(a, sr, bmpk, bfull, sfull, keep, y2full, nrows):
    rpad = (-nrows) % BM
    NR = nrows + rpad
    a2 = jnp.pad(a, (0, rpad))[:, None]
    s2 = jnp.pad(sr, (0, rpad))[:, None]
    bk = jnp.pad(bmpk, ((0, rpad), (0, 0)))
    out = pl.pallas_call(
        _f2_kernel,
        grid=(NR // BM, NP // BN),
        in_specs=[
            pl.BlockSpec((BM, 1), lambda i, j: (i, 0)),
            pl.BlockSpec((BM, 1), lambda i, j: (i, 0)),
            pl.BlockSpec((BM, 128), lambda i, j: (i, j)),
            pl.BlockSpec((1, BN), lambda i, j: (0, j)),
            pl.BlockSpec((1, BN), lambda i, j: (0, j)),
            pl.BlockSpec((1, BN), lambda i, j: (0, j)),
            pl.BlockSpec((BN, 64), lambda i, j: (j, 0)),
        ],
        out_specs=pl.BlockSpec((BM, 128), lambda i, j: (i, 0)),
        out_shape=jax.ShapeDtypeStruct((NR, 128), jnp.float32),
        compiler_params=pltpu.CompilerParams(
            dimension_semantics=("parallel", "arbitrary")),
    )(a2, s2, bk, bfull[None, :], sfull[None, :], keep[None, :], y2full)
    return out[:nrows, 64], out[:nrows, :64]


def _f3_kernel(c_ref, a_ref, s_ref, pr_ref, bm_ref, b_ref, d_ref, sf_ref,
               k_ref, y_ref, o_ref):
    j = pl.program_id(1)
    cc = c_ref[...]
    a = a_ref[...]
    sr = s_ref[...]
    pr = pr_ref[...]
    m = bm_ref[...]
    q = jnp.zeros((BM, 64), jnp.float32)
    p = jnp.zeros((BM, 1), jnp.float32)
    for t in range(16):
        h = jnp.floor(m * 0.5)
        bit = m - 2.0 * h
        m = h
        bt = b_ref[:, t * 128:(t + 1) * 128]
        dt = d_ref[:, t * 128:(t + 1) * 128]
        st = sf_ref[:, t * 128:(t + 1) * 128]
        kt = k_ref[:, t * 128:(t + 1) * 128]
        w = bit * (sr + st) * 0.5
        ak2 = jnp.exp(jnp.maximum(a + bt, 0.0) + w) * pr
        E = kt * jnp.exp(jnp.maximum(cc + dt, 0.0) + ak2)
        q += jnp.dot(E, y_ref[pl.ds(t * 128, 128), :],
                     preferred_element_type=jnp.float32)
        p += jnp.sum(E, axis=1, keepdims=True)
    blk = jnp.concatenate([q, jnp.broadcast_to(p, (BM, 64))], axis=1)

    @pl.when(j == 0)
    def _():
        o_ref[...] = blk

    @pl.when(j > 0)
    def _():
        o_ref[...] += blk


def _flash3(c, a, sr, pinv, bmp2, bfull, dfull, sfull, keep2, y3full, nrows):
    rpad = (-nrows) % BM
    NR = nrows + rpad
    c2 = jnp.pad(c, (0, rpad))[:, None]
    a2 = jnp.pad(a, (0, rpad))[:, None]
    s2 = jnp.pad(sr, (0, rpad))[:, None]
    p2 = jnp.pad(pinv, (0, rpad))[:, None]
    bk = jnp.pad(bmp2, ((0, rpad), (0, 0)))
    out = pl.pallas_call(
        _f3_kernel,
        grid=(NR // BM, NP // BN),
        in_specs=[
            pl.BlockSpec((BM, 1), lambda i, j: (i, 0)),
            pl.BlockSpec((BM, 1), lambda i, j: (i, 0)),
            pl.BlockSpec((BM, 1), lambda i, j: (i, 0)),
            pl.BlockSpec((BM, 1), lambda i, j: (i, 0)),
            pl.BlockSpec((BM, 128), lambda i, j: (i, j)),
            pl.BlockSpec((1, BN), lambda i, j: (0, j)),
            pl.BlockSpec((1, BN), lambda i, j: (0, j)),
            pl.BlockSpec((1, BN), lambda i, j: (0, j)),
            pl.BlockSpec((1, BN), lambda i, j: (0, j)),
            pl.BlockSpec((BN, 64), lambda i, j: (j, 0)),
        ],
        out_specs=pl.BlockSpec((BM, 128), lambda i, j: (i, 0)),
        out_shape=jax.ShapeDtypeStruct((NR, 128), jnp.float32),
        compiler_params=pltpu.CompilerParams(
            dimension_semantics=("parallel", "arbitrary")),
    )(c2, a2, s2, p2, bk, bfull[None, :], dfull[None, :], sfull[None, :],
      keep2[None, :], y3full)
    return out[:nrows, 64], out[:nrows, :64]


def kernel(x, edge_index, W1, b1, W2, b2, W3, b3, att1, att2,
           lin1_w, lin1_b, lin2_w, lin2_b, lin3_w, lin3_b):
    n = x.shape[0]
    row = edge_index[0].astype(jnp.int32)
    col = edge_index[1].astype(jnp.int32)
    s = jnp.sum(x, axis=1)

    T = _build_table(row, col)
    kcol = (jnp.arange(NP, dtype=jnp.int32) < n).astype(jnp.float32)[None, :]

    h0 = _proj(_expmap0(x))
    y1 = _hgc_pre(h0, W1, b1)
    rhs = jnp.pad(jnp.concatenate([y1, s[:, None] * y1], axis=1),
                  ((0, NP - n), (0, 0)))
    o1, dacc, bmp = _pass1(T, rhs, kcol)
    U1 = o1[:n, :64]
    U2 = o1[:n, 64:]
    deg = jnp.sum(dacc, axis=1)[:n]
    sup1 = 0.5 * s[:, None] * U1 + 0.5 * U2
    h1L = _hgc_post(sup1)

    dis = jnp.where(deg > 0, jnp.where(deg > 0, deg, 1.0) ** -0.5, 0.0)
    z = jnp.pad(dis[:, None] * h1L, ((0, NP - n), (0, 0)))
    agg1 = dis[:, None] * _pass2(bmp, z)[:n]
    score1 = jnp.sum(jnp.abs(h1L - agg1), -1)
    k1 = int(np.ceil(0.5 * n))
    _, perm1 = jax.lax.top_k(score1, k1)
    xk1 = h1L[perm1]
    x1 = _readout(xk1)

    sfull = jnp.pad(s, (0, NP - n))
    bfull = jnp.pad(h1L @ att1[64:], (0, NP - n))
    a = xk1 @ att1[:64]
    srow = s[perm1]
    bmpk = bmp[perm1]
    keep = jnp.zeros((NP,), jnp.float32).at[perm1].set(1.0)
    y2full = jnp.pad(_hgc_pre(_proj(_expmap0(h1L)), W2, b2),
                     ((0, NP - n), (0, 0)))
    P, Q = _flash2(a, srow, bmpk, bfull, sfull, keep, y2full, k1)
    h2L = _hgc_post(Q / P[:, None])

    score2 = jnp.sum(jnp.abs(h2L - jnp.mean(h2L, axis=0, keepdims=True)), -1)
    k2 = int(np.ceil(0.5 * k1))
    _, perm2 = jax.lax.top_k(score2, k2)
    xk2 = h2L[perm2]
    x2 = _readout(xk2)

    sel2 = perm1[perm2]
    cvec = xk2 @ att2[:64]
    d = xk2 @ att2[64:]
    dfull = jnp.zeros((NP,), jnp.float32).at[sel2].set(d)
    keep2 = jnp.zeros((NP,), jnp.float32).at[sel2].set(1.0)
    y3 = _hgc_pre(_proj(_expmap0(xk2)), W3, b3)
    y3full = jnp.zeros((NP, 64), jnp.float32).at[sel2].set(y3)
    a2 = a[perm2]
    srow2 = srow[perm2]
    bmp2 = bmpk[perm2]
    pinv = 1.0 / P[perm2]
    P3, Q3 = _flash3(cvec, a2, srow2, pinv, bmp2, bfull, dfull, sfull,
                     keep2, y3full, k2)
    h3L = _hgc_post(Q3 / P3[:, None])
    x3 = _readout(h3L)

    z = jax.nn.relu(x1) + jax.nn.relu(x2) + jax.nn.relu(x3)
    z = jax.nn.relu(z @ lin1_w.T + lin1_b)
    z = jax.nn.relu(z @ lin2_w.T + lin2_b)
    return jax.nn.log_softmax(z @ lin3_w.T + lin3_b, axis=-1)


# speedup vs baseline: 1.8074x; 1.8074x over previous
import functools

import jax
import jax.numpy as jnp
import numpy as np
from jax import lax
from jax.experimental import pallas as pl
from jax.experimental.pallas import tpu as pltpu

C = 1.0
MIN = 1e-15
MAXEPS = 4e-3

NP = 10240
BM = 512
BN = 2048


def _artanh(x):
    return jnp.arctanh(jnp.clip(x, -1 + 1e-7, 1 - 1e-7))


def _proj(x):
    n = jnp.maximum(jnp.linalg.norm(x, axis=-1, keepdims=True), MIN)
    maxnorm = (1.0 - MAXEPS) / jnp.sqrt(C)
    return jnp.where(n > maxnorm, x / n * maxnorm, x)


def _expmap0(u):
    sc = jnp.sqrt(C)
    n = jnp.maximum(jnp.linalg.norm(u, axis=-1, keepdims=True), MIN)
    return jnp.tanh(sc * n) * u / (sc * n)


def _logmap0(p):
    sc = jnp.sqrt(C)
    n = jnp.maximum(jnp.linalg.norm(p, axis=-1, keepdims=True), MIN)
    return _artanh(sc * n) / (sc * n) * p


def _mobius_add(x, y):
    x2 = jnp.sum(x * x, -1, keepdims=True)
    y2 = jnp.sum(y * y, -1, keepdims=True)
    xy = jnp.sum(x * y, -1, keepdims=True)
    num = (1 + 2 * C * xy + C * y2) * x + (1 - C * x2) * y
    den = 1 + 2 * C * xy + C * C * x2 * y2
    return num / jnp.maximum(den, MIN)


def _mobius_matvec(m, x):
    sc = jnp.sqrt(C)
    xn = jnp.maximum(jnp.linalg.norm(x, axis=-1, keepdims=True), MIN)
    mx = x @ m.T
    mxn = jnp.maximum(jnp.linalg.norm(mx, axis=-1, keepdims=True), MIN)
    res = jnp.tanh(mxn / xn * _artanh(sc * xn)) * mx / (mxn * sc)
    zero = jnp.all(mx == 0, axis=-1, keepdims=True)
    return jnp.where(zero, jnp.zeros_like(res), res)


def _hgc_pre(x, W, b):
    mv = _proj(_mobius_matvec(W, x))
    hb = _proj(_expmap0(b))
    h = _proj(_mobius_add(mv, hb))
    return _logmap0(h)


def _hgc_post(sup):
    out = _proj(_expmap0(sup))
    return _logmap0(_proj(_expmap0(jax.nn.relu(_logmap0(out)))))


def _readout(x):
    return jnp.concatenate([jnp.max(x, axis=0, keepdims=True),
                            jnp.mean(x, axis=0, keepdims=True)], axis=1)


def _mm_kernel(a_ref, b_ref, o_ref):
    o_ref[...] = jnp.dot(a_ref[...], b_ref[...],
                         preferred_element_type=jnp.float32)


def _pallas_mm(a, b, bm=256):
    n, k = a.shape
    d = b.shape[1]
    out = pl.pallas_call(
        _mm_kernel,
        grid=(n // bm,),
        in_specs=[
            pl.BlockSpec((bm, k), lambda i: (i, 0)),
            pl.BlockSpec((k, d), lambda i: (0, 0)),
        ],
        out_specs=pl.BlockSpec((bm, d), lambda i: (i, 0)),
        out_shape=jax.ShapeDtypeStruct((n, d), jnp.float32),
    )(a, b)
    return out


def _p1_kernel(t_ref, r_ref, k_ref, o_ref, d_ref, b_ref):
    j = pl.program_id(1)
    blk = t_ref[...]
    o = jnp.dot(blk, r_ref[...], preferred_element_type=jnp.float32)
    acc_d = jnp.zeros((BM, 128), jnp.float32)
    acc_b = jnp.zeros((BM, 128), jnp.float32)
    for t in range(16):
        sl = blk[:, t * 128:(t + 1) * 128]
        acc_d += sl * k_ref[:, t * 128:(t + 1) * 128]
        acc_b += sl * (2.0 ** t)
    b_ref[...] = acc_b

    @pl.when(j == 0)
    def _():
        o_ref[...] = o
        d_ref[...] = acc_d

    @pl.when(j > 0)
    def _():
        o_ref[...] += o
        d_ref[...] += acc_d


def _pass1(T, rhs, kcol):
    return pl.pallas_call(
        _p1_kernel,
        grid=(NP // BM, NP // BN),
        in_specs=[
            pl.BlockSpec((BM, BN), lambda i, j: (i, j)),
            pl.BlockSpec((BN, 128), lambda i, j: (j, 0)),
            pl.BlockSpec((1, BN), lambda i, j: (0, j)),
        ],
        out_specs=[
            pl.BlockSpec((BM, 128), lambda i, j: (i, 0)),
            pl.BlockSpec((BM, 128), lambda i, j: (i, 0)),
            pl.BlockSpec((BM, 128), lambda i, j: (i, j)),
        ],
        out_shape=[
            jax.ShapeDtypeStruct((NP, 128), jnp.float32),
            jax.ShapeDtypeStruct((NP, 128), jnp.float32),
            jax.ShapeDtypeStruct((NP, NP // 16), jnp.float32),
        ],
        compiler_params=pltpu.CompilerParams(
            dimension_semantics=("parallel", "arbitrary")),
    )(T, rhs, kcol)


def _p2_kernel(b_ref, z_ref, o_ref):
    j = pl.program_id(1)
    m = b_ref[...]
    acc = jnp.zeros((BM, 64), jnp.float32)
    for t in range(16):
        h = jnp.floor(m * 0.5)
        bit = m - 2.0 * h
        m = h
        acc += jnp.dot(bit, z_ref[pl.ds(t * 128, 128), :],
                       preferred_element_type=jnp.float32)

    @pl.when(j == 0)
    def _():
        o_ref[...] = acc

    @pl.when(j > 0)
    def _():
        o_ref[...] += acc


def _pass2(bmp, z):
    return pl.pallas_call(
        _p2_kernel,
        grid=(NP // BM, NP // BN),
        in_specs=[
            pl.BlockSpec((BM, 128), lambda i, j: (i, j)),
            pl.BlockSpec((BN, 64), lambda i, j: (j, 0)),
        ],
        out_specs=pl.BlockSpec((BM, 64), lambda i, j: (i, 0)),
        out_shape=jax.ShapeDtypeStruct((NP, 64), jnp.float32),
        compiler_params=pltpu.CompilerParams(
            dimension_semantics=("parallel", "arbitrary")),
    )(bmp, z)


def _f2_kernel(a_ref, s_ref, bm_ref, b_ref, sf_ref, k_ref, y_ref, o_ref):
    j = pl.program_id(1)
    a = a_ref[...]
    sr = s_ref[...]
    m = bm_ref[...]
    q = jnp.zeros((BM, 64), jnp.float32)
    p = jnp.zeros((BM, 1), jnp.float32)
    for t in range(16):
        h = jnp.floor(m * 0.5)
        bit = m - 2.0 * h
        m = h
        bt = b_ref[:, t * 128:(t + 1) * 128]
        st = sf_ref[:, t * 128:(t + 1) * 128]
        kt = k_ref[:, t * 128:(t + 1) * 128]
        w = bit * (sr + st) * 0.5
        E = kt * jnp.exp(jnp.maximum(a + bt, 0.0) + w)
        q += jnp.dot(E, y_ref[pl.ds(t * 128, 128), :],
                     preferred_element_type=jnp.float32)
        p += jnp.sum(E, axis=1, keepdims=True)
    blk = jnp.concatenate([q, jnp.broadcast_to(p, (BM, 64))], axis=1)

    @pl.when(j == 0)
    def _():
        o_ref[...] = blk

    @pl.when(j > 0)
    def _():
        o_ref[...] += blk


def _flash2(a, sr, bmpk, bfull, sfull, keep, y2full, nrows):
    rpad = (-nrows) % BM
    NR = nrows + rpad
    a2 = jnp.pad(a, (0, rpad))[:, None]
    s2 = jnp.pad(sr, (0, rpad))[:, None]
    bk = jnp.pad(bmpk, ((0, rpad), (0, 0)))
    out = pl.pallas_call(
        _f2_kernel,
        grid=(NR // BM, NP // BN),
        in_specs=[
            pl.BlockSpec((BM, 1), lambda i, j: (i, 0)),
            pl.BlockSpec((BM, 1), lambda i, j: (i, 0)),
            pl.BlockSpec((BM, 128), lambda i, j: (i, j)),
            pl.BlockSpec((1, BN), lambda i, j: (0, j)),
            pl.BlockSpec((1, BN), lambda i, j: (0, j)),
            pl.BlockSpec((1, BN), lambda i, j: (0, j)),
            pl.BlockSpec((BN, 64), lambda i, j: (j, 0)),
        ],
        out_specs=pl.BlockSpec((BM, 128), lambda i, j: (i, 0)),
        out_shape=jax.ShapeDtypeStruct((NR, 128), jnp.float32),
        compiler_params=pltpu.CompilerParams(
            dimension_semantics=("parallel", "arbitrary")),
    )(a2, s2, bk, bfull[None, :], sfull[None, :], keep[None, :], y2full)
    return out[:nrows, 64], out[:nrows, :64]


def _f3_kernel(c_ref, a_ref, s_ref, pr_ref, bm_ref, b_ref, d_ref, sf_ref,
               k_ref, y_ref, o_ref):
    j = pl.program_id(1)
    cc = c_ref[...]
    a = a_ref[...]
    sr = s_ref[...]
    pr = pr_ref[...]
    m = bm_ref[...]
    q = jnp.zeros((BM, 64), jnp.float32)
    p = jnp.zeros((BM, 1), jnp.float32)
    for t in range(16):
        h = jnp.floor(m * 0.5)
        bit = m - 2.0 * h
        m = h
        bt = b_ref[:, t * 128:(t + 1) * 128]
        dt = d_ref[:, t * 128:(t + 1) * 128]
        st = sf_ref[:, t * 128:(t + 1) * 128]
        kt = k_ref[:, t * 128:(t + 1) * 128]
        w = bit * (sr + st) * 0.5
        ak2 = jnp.exp(jnp.maximum(a + bt, 0.0) + w) * pr
        E = kt * jnp.exp(jnp.maximum(cc + dt, 0.0) + ak2)
        q += jnp.dot(E, y_ref[pl.ds(t * 128, 128), :],
                     preferred_element_type=jnp.float32)
        p += jnp.sum(E, axis=1, keepdims=True)
    blk = jnp.concatenate([q, jnp.broadcast_to(p, (BM, 64))], axis=1)

    @pl.when(j == 0)
    def _():
        o_ref[...] = blk

    @pl.when(j > 0)
    def _():
        o_ref[...] += blk


def _flash3(c, a, sr, pinv, bmp2, bfull, dfull, sfull, keep2, y3full, nrows):
    rpad = (-nrows) % BM
    NR = nrows + rpad
    c2 = jnp.pad(c, (0, rpad))[:, None]
    a2 = jnp.pad(a, (0, rpad))[:, None]
    s2 = jnp.pad(sr, (0, rpad))[:, None]
    p2 = jnp.pad(pinv, (0, rpad))[:, None]
    bk = jnp.pad(bmp2, ((0, rpad), (0, 0)))
    out = pl.pallas_call(
        _f3_kernel,
        grid=(NR // BM, NP // BN),
        in_specs=[
            pl.BlockSpec((BM, 1), lambda i, j: (i, 0)),
            pl.BlockSpec((BM, 1), lambda i, j: (i, 0)),
            pl.BlockSpec((BM, 1), lambda i, j: (i, 0)),
            pl.BlockSpec((BM, 1), lambda i, j: (i, 0)),
            pl.BlockSpec((BM, 128), lambda i, j: (i, j)),
            pl.BlockSpec((1, BN), lambda i, j: (0, j)),
            pl.BlockSpec((1, BN), lambda i, j: (0, j)),
            pl.BlockSpec((1, BN), lambda i, j: (0, j)),
            pl.BlockSpec((1, BN), lambda i, j: (0, j)),
            pl.BlockSpec((BN, 64), lambda i, j: (j, 0)),
        ],
        out_specs=pl.BlockSpec((BM, 128), lambda i, j: (i, 0)),
        out_shape=jax.ShapeDtypeStruct((NR, 128), jnp.float32),
        compiler_params=pltpu.CompilerParams(
            dimension_semantics=("parallel", "arbitrary")),
    )(c2, a2, s2, p2, bk, bfull[None, :], dfull[None, :], sfull[None, :],
      keep2[None, :], y3full)
    return out[:nrows, 64], out[:nrows, :64]


def kernel(x, edge_index, W1, b1, W2, b2, W3, b3, att1, att2,
           lin1_w, lin1_b, lin2_w, lin2_b, lin3_w, lin3_b):
    n = x.shape[0]
    row = edge_index[0].astype(jnp.int32)
    col = edge_index[1].astype(jnp.int32)
    s = jnp.sum(x, axis=1)

    T = jnp.zeros((NP * NP,), jnp.float32).at[row * NP + col].max(
        1.0, mode="promise_in_bounds").reshape(NP, NP)
    kcol = (jnp.arange(NP, dtype=jnp.int32) < n).astype(jnp.float32)[None, :]

    h0 = _proj(_expmap0(x))
    y1 = _hgc_pre(h0, W1, b1)
    rhs = jnp.pad(jnp.concatenate([y1, s[:, None] * y1], axis=1),
                  ((0, NP - n), (0, 0)))
    o1, dacc, bmp = _pass1(T, rhs, kcol)
    U1 = o1[:n, :64]
    U2 = o1[:n, 64:]
    deg = jnp.sum(dacc, axis=1)[:n]
    sup1 = 0.5 * s[:, None] * U1 + 0.5 * U2
    h1L = _hgc_post(sup1)

    dis = jnp.where(deg > 0, jnp.where(deg > 0, deg, 1.0) ** -0.5, 0.0)
    z = jnp.pad(dis[:, None] * h1L, ((0, NP - n), (0, 0)))
    agg1 = dis[:, None] * _pass2(bmp, z)[:n]
    score1 = jnp.sum(jnp.abs(h1L - agg1), -1)
    k1 = int(np.ceil(0.5 * n))
    _, perm1 = jax.lax.top_k(score1, k1)
    xk1 = h1L[perm1]
    x1 = _readout(xk1)

    sfull = jnp.pad(s, (0, NP - n))
    bfull = jnp.pad(h1L @ att1[64:], (0, NP - n))
    a = xk1 @ att1[:64]
    srow = s[perm1]
    bmpk = bmp[perm1]
    keep = jnp.zeros((NP,), jnp.float32).at[perm1].set(1.0)
    y2full = jnp.pad(_hgc_pre(_proj(_expmap0(h1L)), W2, b2),
                     ((0, NP - n), (0, 0)))
    P, Q = _flash2(a, srow, bmpk, bfull, sfull, keep, y2full, k1)
    h2L = _hgc_post(Q / P[:, None])

    score2 = jnp.sum(jnp.abs(h2L - jnp.mean(h2L, axis=0, keepdims=True)), -1)
    k2 = int(np.ceil(0.5 * k1))
    _, perm2 = jax.lax.top_k(score2, k2)
    xk2 = h2L[perm2]
    x2 = _readout(xk2)

    sel2 = perm1[perm2]
    cvec = xk2 @ att2[:64]
    d = xk2 @ att2[64:]
    dfull = jnp.zeros((NP,), jnp.float32).at[sel2].set(d)
    keep2 = jnp.zeros((NP,), jnp.float32).at[sel2].set(1.0)
    y3 = _hgc_pre(_proj(_expmap0(xk2)), W3, b3)
    y3full = jnp.zeros((NP, 64), jnp.float32).at[sel2].set(y3)
    a2 = a[perm2]
    srow2 = srow[perm2]
    bmp2 = bmpk[perm2]
    pinv = 1.0 / P[perm2]
    P3, Q3 = _flash3(cvec, a2, srow2, pinv, bmp2, bfull, dfull, sfull,
                     keep2, y3full, k2)
    h3L = _hgc_post(Q3 / P3[:, None])
    x3 = _readout(h3L)

    z = jax.nn.relu(x1) + jax.nn.relu(x2) + jax.nn.relu(x3)
    z = jax.nn.relu(z @ lin1_w.T + lin1_b)
    z = jax.nn.relu(z @ lin2_w.T + lin2_b)
    return jax.nn.log_softmax(z @ lin3_w.T + lin3_b, axis=-1)
